# baseline (speedup 1.0000x reference)
import functools

import jax
import jax.numpy as jnp
from jax import lax
from jax.experimental import pallas as pl
from jax.experimental.pallas import tpu as pltpu
from jax.experimental.pallas import tpu_sc as plsc

_B = 8192
_K = 8192
_D = 256

_BM = 1024
_BN = 8192

_NC = 2
_NS = 16
_NW = _NC * _NS
_BPW = _B // _NW


_C = 128


def _argmin_body(z_ref, w_ref, idx_ref, loss_ref, esq_s, acc):
    z2 = z_ref[...] * -2.0

    nch = _K // _C

    @pl.when(pl.program_id(0) == 0)
    def _():
        for c in range(_K // _C):
            esq_s[c:c + 1, :] = jnp.sum(
                w_ref[c * _C:(c + 1) * _C, :] ** 2, axis=1).reshape(1, _C)

    zb = jnp.sum(z_ref[...] ** 2, axis=1, keepdims=True)

    def fold2(a, b):
        take = b[0] < a[0]
        return (jnp.minimum(a[0], b[0]), jnp.where(take, b[1], a[1]))

    stack = []
    qw = 4096
    for q in range(_K // qw):
        m2q = lax.dot_general(z2, w_ref[q * qw:(q + 1) * qw, :],
                              (((1,), (1,)), ((), ())),
                              preferred_element_type=jnp.float32)
        for cc in range(qw // _C):
            c = q * (qw // _C) + cc
            sl = slice(cc * _C, (cc + 1) * _C)
            node = ((zb + esq_s[c:c + 1, :]) + m2q[:, sl], jnp.float32(c))
            d = 1
            while stack and stack[-1][0] == d:
                node = fold2(stack.pop()[1], node)
                d *= 2
            stack.append((d, node))
    bval, bid = stack[0][1]

    lane = lax.broadcasted_iota(jnp.int32, (_BM, _C), 1).astype(jnp.float32)
    gindex = bid * float(_C) + lane
    gmin = jnp.min(bval, axis=1, keepdims=True)
    gidx = jnp.min(jnp.where(bval == gmin, gindex, 3.4e38), axis=1,
                   keepdims=True)
    idx_ref[...] = gidx.astype(jnp.int32)
    i = pl.program_id(0)
    prev = jnp.where(i == 0, 0.0, acc[0])
    acc[0] = prev + jnp.sum(gmin)

    @pl.when(i == pl.num_programs(0) - 1)
    def _():
        loss_ref[...] = jnp.broadcast_to(acc[0] * (1.0 / 2097152.0), (1, 1))


def _distance_argmin(z_e, W, interpret=False):
    rows = z_e.shape[0]
    return pl.pallas_call(
        _argmin_body,
        grid=(rows // _BM,),
        in_specs=[
            pl.BlockSpec((_BM, _D), lambda i: (i, 0)),
            pl.BlockSpec((_K, _D), lambda i: (0, 0)),
        ],
        out_specs=[
            pl.BlockSpec((_BM, 1), lambda i: (i, 0)),
            pl.BlockSpec((1, 1), lambda i: (0, 0)),
        ],
        out_shape=[
            jax.ShapeDtypeStruct((rows, 1), jnp.int32),
            jax.ShapeDtypeStruct((1, 1), jnp.float32),
        ],
        scratch_shapes=[
            pltpu.VMEM((_K // _C, _C), jnp.float32),
            pltpu.SMEM((1,), jnp.float32),
        ],
        compiler_params=pltpu.CompilerParams(
            dimension_semantics=("arbitrary",)),
        interpret=interpret,
    )(z_e, W)


def _sc_gather(W, idx):
    n = idx.shape[0]
    bpw = n // _NW
    mesh = plsc.VectorSubcoreMesh(core_axis_name="c", subcore_axis_name="s",
                                  num_cores=_NC, num_subcores=_NS)

    @functools.partial(
        pl.kernel,
        out_type=jax.ShapeDtypeStruct((n, _D), jnp.float32),
        mesh=mesh,
        scratch_types=[
            pltpu.VMEM((bpw,), jnp.int32),
            pltpu.VMEM((bpw, _D), jnp.float32),
            pltpu.SemaphoreType.DMA,
        ],
    )
    def k(table_hbm, idx_hbm, out_hbm, idx_v, rows_v, sem):
        wid = lax.axis_index("s") * _NC + lax.axis_index("c")
        base = wid * bpw
        pltpu.sync_copy(idx_hbm.at[pl.ds(base, bpw)], idx_v)
        pltpu.async_copy(table_hbm.at[idx_v], rows_v, sem).wait()
        pltpu.sync_copy(rows_v, out_hbm.at[pl.ds(base, bpw)])

    return k(W, idx)


def kernel(z_e, W):
    h = _B // 2
    idx2a, loss2a = _distance_argmin(z_e[:h], W)
    idxa = idx2a.reshape(h)
    zqa = _sc_gather(W, idxa)
    idx2b, loss2b = _distance_argmin(z_e[h:], W)
    idxb = idx2b.reshape(h)
    zqb = _sc_gather(W, idxb)
    z_q_st = jnp.concatenate([zqa, zqb], axis=0)
    loss = loss2a[0, 0] + loss2b[0, 0]
    idx = jnp.concatenate([idxa, idxb], axis=0)
    return (z_q_st, loss, loss, idx)


# speedup vs baseline: 1.2327x; 1.2327x over previous
import functools

import jax
import jax.numpy as jnp
from jax import lax
from jax.experimental import pallas as pl
from jax.experimental.pallas import tpu as pltpu
from jax.experimental.pallas import tpu_sc as plsc

_B = 8192
_K = 8192
_D = 256

_BM = 2048
_BN = 8192

_NC = 2
_NS = 16
_NW = _NC * _NS
_BPW = _B // _NW


_C = 128


def _argmin_body(z_ref, w_ref, idx_ref, loss_ref, esq_s, acc):
    z2 = z_ref[...] * -2.0

    nch = _K // _C

    @pl.when(pl.program_id(0) == 0)
    def _():
        for c in range(_K // _C):
            esq_s[c:c + 1, :] = jnp.sum(
                w_ref[c * _C:(c + 1) * _C, :] ** 2, axis=1).reshape(1, _C)

    zb = jnp.sum(z_ref[...] ** 2, axis=1, keepdims=True)

    def fold2(a, b):
        take = b[0] < a[0]
        return (jnp.minimum(a[0], b[0]), jnp.where(take, b[1], a[1]))

    stack = []
    qw = 2048
    for q in range(_K // qw):
        m2q = lax.dot_general(z2, w_ref[q * qw:(q + 1) * qw, :],
                              (((1,), (1,)), ((), ())),
                              preferred_element_type=jnp.float32)
        for cc in range(qw // _C):
            c = q * (qw // _C) + cc
            sl = slice(cc * _C, (cc + 1) * _C)
            node = ((zb + esq_s[c:c + 1, :]) + m2q[:, sl], jnp.float32(c))
            d = 1
            while stack and stack[-1][0] == d:
                node = fold2(stack.pop()[1], node)
                d *= 2
            stack.append((d, node))
    bval, bid = stack[0][1]

    lane = lax.broadcasted_iota(jnp.int32, (_BM, _C), 1).astype(jnp.float32)
    gindex = bid * float(_C) + lane
    gmin = jnp.min(bval, axis=1, keepdims=True)
    gidx = jnp.min(jnp.where(bval == gmin, gindex, 3.4e38), axis=1,
                   keepdims=True)
    idx_ref[...] = gidx.astype(jnp.int32)
    i = pl.program_id(0)
    prev = jnp.where(i == 0, 0.0, acc[0])
    acc[0] = prev + jnp.sum(gmin)

    @pl.when(i == pl.num_programs(0) - 1)
    def _():
        loss_ref[...] = jnp.broadcast_to(acc[0] * (1.0 / 2097152.0), (1, 1))


def _distance_argmin(z_e, W, interpret=False):
    return pl.pallas_call(
        _argmin_body,
        grid=(_B // _BM,),
        in_specs=[
            pl.BlockSpec((_BM, _D), lambda i: (i, 0)),
            pl.BlockSpec((_K, _D), lambda i: (0, 0)),
        ],
        out_specs=[
            pl.BlockSpec((_BM, 1), lambda i: (i, 0)),
            pl.BlockSpec((1, 1), lambda i: (0, 0)),
        ],
        out_shape=[
            jax.ShapeDtypeStruct((_B, 1), jnp.int32),
            jax.ShapeDtypeStruct((1, 1), jnp.float32),
        ],
        scratch_shapes=[
            pltpu.VMEM((_K // _C, _C), jnp.float32),
            pltpu.SMEM((1,), jnp.float32),
        ],
        compiler_params=pltpu.CompilerParams(
            dimension_semantics=("arbitrary",)),
        interpret=interpret,
    )(z_e, W)


def _sc_gather(W, idx):
    mesh = plsc.VectorSubcoreMesh(core_axis_name="c", subcore_axis_name="s",
                                  num_cores=_NC, num_subcores=_NS)

    @functools.partial(
        pl.kernel,
        out_type=jax.ShapeDtypeStruct((_B, _D), jnp.float32),
        mesh=mesh,
        scratch_types=[
            pltpu.VMEM((_BPW,), jnp.int32),
            pltpu.VMEM((_BPW, _D), jnp.float32),
            pltpu.SemaphoreType.DMA,
        ],
    )
    def k(table_hbm, idx_hbm, out_hbm, idx_v, rows_v, sem):
        wid = lax.axis_index("s") * _NC + lax.axis_index("c")
        base = wid * _BPW
        pltpu.sync_copy(idx_hbm.at[pl.ds(base, _BPW)], idx_v)
        pltpu.async_copy(table_hbm.at[idx_v], rows_v, sem).wait()
        pltpu.sync_copy(rows_v, out_hbm.at[pl.ds(base, _BPW)])

    return k(W, idx)


def kernel(z_e, W):
    idx2d, loss2d = _distance_argmin(z_e, W)
    idx = idx2d.reshape(_B)
    z_q_st = _sc_gather(W, idx)
    loss = loss2d[0, 0]
    return (z_q_st, loss, loss, idx)


# speedup vs baseline: 1.2426x; 1.0080x over previous
import functools

import jax
import jax.numpy as jnp
from jax import lax
from jax.experimental import pallas as pl
from jax.experimental.pallas import tpu as pltpu
from jax.experimental.pallas import tpu_sc as plsc

_B = 8192
_K = 8192
_D = 256

_BM = 4096
_BN = 8192

_NC = 2
_NS = 16
_NW = _NC * _NS
_BPW = _B // _NW


_C = 128


def _argmin_body(z_ref, w_ref, idx_ref, loss_ref, esq_s, acc):
    z2 = z_ref[...] * -2.0

    nch = _K // _C

    @pl.when(pl.program_id(0) == 0)
    def _():
        for c in range(_K // _C):
            esq_s[c:c + 1, :] = jnp.sum(
                w_ref[c * _C:(c + 1) * _C, :] ** 2, axis=1).reshape(1, _C)

    zb = jnp.sum(z_ref[...] ** 2, axis=1, keepdims=True)

    def fold2(a, b):
        take = b[0] < a[0]
        return (jnp.minimum(a[0], b[0]), jnp.where(take, b[1], a[1]))

    stack = []
    qw = 1024
    for q in range(_K // qw):
        m2q = lax.dot_general(z2, w_ref[q * qw:(q + 1) * qw, :],
                              (((1,), (1,)), ((), ())),
                              preferred_element_type=jnp.float32)
        for cc in range(qw // _C):
            c = q * (qw // _C) + cc
            sl = slice(cc * _C, (cc + 1) * _C)
            node = ((zb + esq_s[c:c + 1, :]) + m2q[:, sl], jnp.float32(c))
            d = 1
            while stack and stack[-1][0] == d:
                node = fold2(stack.pop()[1], node)
                d *= 2
            stack.append((d, node))
    bval, bid = stack[0][1]

    lane = lax.broadcasted_iota(jnp.int32, (_BM, _C), 1).astype(jnp.float32)
    gindex = bid * float(_C) + lane
    gmin = jnp.min(bval, axis=1, keepdims=True)
    gidx = jnp.min(jnp.where(bval == gmin, gindex, 3.4e38), axis=1,
                   keepdims=True)
    idx_ref[...] = gidx.astype(jnp.int32)
    i = pl.program_id(0)
    prev = jnp.where(i == 0, 0.0, acc[0])
    acc[0] = prev + jnp.sum(gmin)

    @pl.when(i == pl.num_programs(0) - 1)
    def _():
        loss_ref[...] = jnp.broadcast_to(acc[0] * (1.0 / 2097152.0), (1, 1))


def _distance_argmin(z_e, W, interpret=False):
    return pl.pallas_call(
        _argmin_body,
        grid=(_B // _BM,),
        in_specs=[
            pl.BlockSpec((_BM, _D), lambda i: (i, 0)),
            pl.BlockSpec((_K, _D), lambda i: (0, 0)),
        ],
        out_specs=[
            pl.BlockSpec((_BM, 1), lambda i: (i, 0)),
            pl.BlockSpec((1, 1), lambda i: (0, 0)),
        ],
        out_shape=[
            jax.ShapeDtypeStruct((_B, 1), jnp.int32),
            jax.ShapeDtypeStruct((1, 1), jnp.float32),
        ],
        scratch_shapes=[
            pltpu.VMEM((_K // _C, _C), jnp.float32),
            pltpu.SMEM((1,), jnp.float32),
        ],
        compiler_params=pltpu.CompilerParams(
            dimension_semantics=("arbitrary",)),
        interpret=interpret,
    )(z_e, W)


def _sc_gather(W, idx):
    mesh = plsc.VectorSubcoreMesh(core_axis_name="c", subcore_axis_name="s",
                                  num_cores=_NC, num_subcores=_NS)

    @functools.partial(
        pl.kernel,
        out_type=jax.ShapeDtypeStruct((_B, _D), jnp.float32),
        mesh=mesh,
        scratch_types=[
            pltpu.VMEM((_BPW,), jnp.int32),
            pltpu.VMEM((_BPW, _D), jnp.float32),
            pltpu.SemaphoreType.DMA,
        ],
    )
    def k(table_hbm, idx_hbm, out_hbm, idx_v, rows_v, sem):
        wid = lax.axis_index("s") * _NC + lax.axis_index("c")
        base = wid * _BPW
        pltpu.sync_copy(idx_hbm.at[pl.ds(base, _BPW)], idx_v)
        pltpu.async_copy(table_hbm.at[idx_v], rows_v, sem).wait()
        pltpu.sync_copy(rows_v, out_hbm.at[pl.ds(base, _BPW)])

    return k(W, idx)


def kernel(z_e, W):
    idx2d, loss2d = _distance_argmin(z_e, W)
    idx = idx2d.reshape(_B)
    z_q_st = _sc_gather(W, idx)
    loss = loss2d[0, 0]
    return (z_q_st, loss, loss, idx)
